# Initial kernel scaffold; baseline (speedup 1.0000x reference)
import jax
import jax.numpy as jnp
from jax.experimental import pallas as pl


def kernel(signal, window):
    raise NotImplementedError("write your pallas kernel here")


# speedup vs baseline: 863.4830x; 863.4830x over previous
import functools

import numpy as np
import jax
import jax.numpy as jnp
from jax import lax
from jax.experimental import pallas as pl
from jax.experimental.pallas import tpu as pltpu
from jax.experimental.pallas import tpu_sc as plsc

_N_FFT = 1024
_HOP = 256
_SR = 44100
_SIG_LEN = 2097252
_NFRAMES = 8193
_NBF = 513
_NBT = 8193
_FB = 128
_NBLK = 65
_FPAD = _NBLK * _FB
_LANES = 640
_CW = 128
_NCH = 65
_HSZ = _NBF * _CW
_SROWS = 22
_NVEC = 33

_T_HI = _NBT * _HOP / _SR
_WT = _T_HI / _NBT
_WF = 0.5 / _NBF
_WIN_DUR = _N_FFT / _SR
_F32 = np.float32


def _dft_mat(anchor):
    eye = jnp.eye(_N_FFT, dtype=jnp.float32) * (anchor * 0 + 1)
    tz = jnp.fft.rfft(eye, axis=1)
    pad = ((0, 0), (0, _LANES - _NBF))
    tr = jnp.pad(jnp.real(tz).astype(jnp.float32), pad)
    ti = jnp.pad(jnp.imag(tz).astype(jnp.float32), pad)
    return jnp.concatenate([tr, ti], axis=1)


def _tc_points_body(xp_ref, xs_ref, win_ref, cen_ref, cs_ref, out_ref):
    b = pl.program_id(0)

    def frames_of(ref):
        x = ref[pl.ds(b * _FB, _FB + 3), :]
        fr = jnp.concatenate(
            [x[0:_FB], x[1:_FB + 1], x[2:_FB + 2], x[3:_FB + 3]], axis=1)
        return fr * win_ref[...]

    def spec_of(ref):
        rei = lax.dot_general(
            frames_of(ref), cs_ref[...], (((1,), (0,)), ((), ())),
            precision=lax.Precision.HIGHEST,
            preferred_element_type=jnp.float32)
        return rei[:, :_LANES], rei[:, _LANES:]

    re, im = spec_of(xp_ref)
    rt, it = spec_of(xs_ref)

    inv2pi_den = _F32(2.0 * np.pi)

    ct_re = re * rt + im * it
    ct_im = im * rt - re * it
    f = jnp.mod(jnp.arctan2(ct_im, ct_re) / inv2pi_den, _F32(1.0))

    rp = jnp.roll(re, 1, axis=1)
    ip = jnp.roll(im, 1, axis=1)
    cf_re = re * rp + im * ip
    cf_im = im * rp - re * ip
    af = jnp.mod(jnp.arctan2(cf_im, cf_re) / inv2pi_den, _F32(1.0))
    lane = lax.broadcasted_iota(jnp.int32, (_FB, _LANES), 1)
    af = jnp.where(lane == 0, _F32(0.0), af)
    delay = _F32(0.5) - af

    t = cen_ref[...] + delay * _F32(_WIN_DUR)

    w = jnp.sqrt(re * re + im * im) / _F32(_NBF)
    inb = (f >= _F32(0.0)) & (f <= _F32(0.5)) \
        & (t >= _F32(0.0)) & (t <= _F32(_T_HI))

    fi = jnp.clip(jnp.floor(f / _F32(_WF)).astype(jnp.int32), 0, _NBF - 1)
    ti = jnp.clip(jnp.floor(t / _F32(_WT)).astype(jnp.int32), 0, _NBT - 1)

    j = b * _FB + lax.broadcasted_iota(jnp.int32, (_FB, _LANES), 0)
    dtc = jnp.clip(ti - j + 1, 0, 7)
    wfin = jnp.where(inb & (j < _NFRAMES), w, _F32(0.0))

    wu = lax.bitcast_convert_type(wfin.astype(jnp.bfloat16), jnp.uint16)
    packed = (wu.astype(jnp.int32) << 16) | (fi * 8 + dtc)
    out_ref[...] = packed


def _tc_points(xp2d, xs2d, win2d, cen2d, csmat):
    return pl.pallas_call(
        _tc_points_body,
        grid=(_NBLK,),
        in_specs=[
            pl.BlockSpec(xp2d.shape, lambda b: (0, 0)),
            pl.BlockSpec(xs2d.shape, lambda b: (0, 0)),
            pl.BlockSpec(win2d.shape, lambda b: (0, 0)),
            pl.BlockSpec((_FB, 1), lambda b: (b, 0)),
            pl.BlockSpec((_N_FFT, 2 * _LANES), lambda b: (0, 0)),
        ],
        out_specs=pl.BlockSpec((_FB, _LANES), lambda b: (b, 0)),
        out_shape=jax.ShapeDtypeStruct((_FPAD, _LANES), jnp.int32),
    )(xp2d, xs2d, win2d, cen2d, csmat)


def _sc_hist_kernel(points_hbm, zeros_hbm, out_hbm, stage_v, hist_v, sem):
    del sem
    wid = lax.axis_index("s") * 2 + lax.axis_index("c")

    for kk in range(3):
        c = wid + kk * 32

        @pl.when(c < _NCH)
        def _():
            pltpu.sync_copy(zeros_hbm, hist_v)
            row0 = jnp.maximum(c * _CW - 4, 0)
            colbase = row0 - c * _CW - 1
            for sub in range(6):
                pltpu.sync_copy(
                    points_hbm.at[pl.ds((row0 + sub * _SROWS) * _LANES,
                                        _SROWS * _LANES)],
                    stage_v)

                def row_body(rr, _):
                    def vec_body(v, __):
                        pv = stage_v[pl.ds(rr * _LANES + v * 16, 16)]
                        idxl = pv & 0xFFFF
                        wvec = lax.bitcast_convert_type(
                            pv & jnp.int32(-65536), jnp.float32)
                        fi = idxl >> 3
                        dtf = idxl & 7
                        col = dtf + (colbase + sub * _SROWS + rr)
                        valid = (col >= 0) & (col < _CW)
                        lidx = fi * _CW + jnp.clip(col, 0, _CW - 1)
                        plsc.addupdate_scatter(hist_v, [lidx], wvec,
                                               mask=valid)
                        return __
                    return lax.fori_loop(0, _NVEC, vec_body, _)
                lax.fori_loop(0, _SROWS, row_body, 0)
            pltpu.sync_copy(hist_v, out_hbm.at[c])


def _sc_hist(points_flat, zeros):
    mesh = plsc.VectorSubcoreMesh(core_axis_name="c", subcore_axis_name="s")
    fn = functools.partial(
        pl.kernel,
        mesh=mesh,
        compiler_params=pltpu.CompilerParams(needs_layout_passes=False),
        out_type=jax.ShapeDtypeStruct((_NCH, _HSZ), jnp.float32),
        scratch_types=[
            pltpu.VMEM((_SROWS * _LANES,), jnp.int32),
            pltpu.VMEM((_HSZ,), jnp.float32),
            pltpu.SemaphoreType.DMA,
        ],
    )(_sc_hist_kernel)
    return fn(points_flat, zeros)


def _tc_log_body(h_ref, out_ref):
    h = h_ref[0]
    hm = jnp.maximum(_F32(1e-6), h)
    out_ref[...] = _F32(20.0) * (jnp.log(hm) / jnp.log(_F32(10.0)))


def _tc_log(hblocks):
    return pl.pallas_call(
        _tc_log_body,
        grid=(_NCH,),
        in_specs=[pl.BlockSpec((1, _NBF, _CW), lambda c: (c, 0, 0))],
        out_specs=pl.BlockSpec((_NBF, _CW), lambda c: (0, c)),
        out_shape=jax.ShapeDtypeStruct((_NBF, _NBT), jnp.float32),
    )(hblocks)


def kernel(signal, window):
    pad = _N_FFT // 2
    xp = jnp.pad(signal, (pad, pad), mode='reflect')
    ts = jnp.roll(signal, 1).at[0].set(0.0)
    xs = jnp.pad(ts, (pad, pad), mode='reflect')

    rows = _FPAD + 8
    total = rows * _HOP
    xp2d = jnp.pad(xp, (0, total - xp.shape[0])).reshape(rows, _HOP)
    xs2d = jnp.pad(xs, (0, total - xs.shape[0])).reshape(rows, _HOP)
    win2d = window.reshape(1, _N_FFT)

    duration = signal.shape[0] / _SR
    win_starts = jnp.arange(0.0, duration, _HOP / _SR)
    eps = float(np.finfo(np.float32).eps)
    centers = win_starts + _WIN_DUR / 2 + eps
    cen2d = jnp.pad(centers, (0, _FPAD - _NFRAMES)).reshape(_FPAD, 1)
    cen2d = cen2d.astype(jnp.float32)

    csmat = _dft_mat(signal[0])
    points = _tc_points(xp2d, xs2d, win2d, cen2d, csmat)
    points_flat = points.reshape(-1)

    zeros = jnp.zeros((_HSZ,), jnp.float32)
    hist = _sc_hist(points_flat, zeros)

    return _tc_log(hist.reshape(_NCH, _NBF, _CW))
